# baseline (speedup 1.0000x reference)
import functools

import jax
import jax.numpy as jnp
from jax import lax
from jax.experimental import pallas as pl
from jax.experimental.pallas import tpu as pltpu
from jax.experimental.pallas import tpu_sc as plsc

N = 10000
E = 320000
D = 128
OUT = 128

NC = 2
NS = 16
NT = NC * NS
CHUNK = 128
KJ = 80
NH = 2
HKJ = KJ // NH
EPAD = NT * KJ * CHUNK
NPAD = 10112
RPT = NPAD // NS

BN = 400
NB = N // BN

_mesh = plsc.VectorSubcoreMesh(core_axis_name="c", subcore_axis_name="s")


@functools.partial(
    pl.kernel,
    mesh=_mesh,
    out_type=jax.ShapeDtypeStruct((NC, NPAD, 16), jnp.float32),
    scratch_types=[
        pltpu.VMEM((KJ, CHUNK), jnp.int32),
        pltpu.VMEM((CHUNK, 16), jnp.float32),
        pltpu.VMEM_SHARED((NPAD, 16), jnp.float32),
    ],
)
def _deg_count(dst_hbm, zd_hbm, ones_hbm, d_out, idx_d, ones_v, d_sh):
    c = lax.axis_index("c")
    s = lax.axis_index("s")
    wid = c * NS + s
    pltpu.sync_copy(zd_hbm, d_sh.at[pl.ds(s * RPT, RPT)])
    pltpu.sync_copy(ones_hbm, ones_v)
    pltpu.sync_copy(dst_hbm.at[wid], idx_d)
    plsc.subcore_barrier()

    def body(j, carry):
        pltpu.sync_copy(ones_v, d_sh.at[idx_d.at[j]], add=True)
        return carry

    lax.fori_loop(0, KJ, body, 0)
    plsc.subcore_barrier()
    pltpu.sync_copy(d_sh.at[pl.ds(s * RPT, RPT)], d_out.at[c, pl.ds(s * RPT, RPT)])


@functools.partial(
    pl.kernel,
    mesh=_mesh,
    out_type=jax.ShapeDtypeStruct((NC, NPAD, D), jnp.float32),
    scratch_types=[
        pltpu.VMEM((KJ, CHUNK), jnp.int32),
        pltpu.VMEM((KJ, CHUNK), jnp.int32),
        pltpu.VMEM((CHUNK, D), jnp.float32),
        pltpu.VMEM_SHARED((NPAD, D), jnp.float32),
        pltpu.SemaphoreType.DMA,
    ],
)
def _seg_sum(x_hbm, src_hbm, dst_hbm, zs_hbm, s_out,
             idx_s, idx_d, rows, s_sh, sem):
    c = lax.axis_index("c")
    s = lax.axis_index("s")
    wid = c * NS + s

    pltpu.sync_copy(zs_hbm, s_sh.at[pl.ds(s * RPT, RPT)])
    pltpu.sync_copy(src_hbm.at[wid], idx_s)
    pltpu.sync_copy(dst_hbm.at[wid], idx_d)
    plsc.subcore_barrier()

    def body(j, carry):
        pltpu.async_copy(x_hbm.at[idx_s.at[j]], rows, sem).wait()
        pltpu.sync_copy(rows, s_sh.at[idx_d.at[j]], add=True)
        return carry

    lax.fori_loop(0, KJ, body, 0)
    plsc.subcore_barrier()
    pltpu.sync_copy(s_sh.at[pl.ds(s * RPT, RPT)], s_out.at[c, pl.ds(s * RPT, RPT)])


def _layer_body(x_ref, s_ref, deg_ref, wm_ref, bm_ref, wa_ref, ba_ref,
                wb_ref, bb_ref, out_ref):
    agg = s_ref[0] + s_ref[1]
    deg = deg_ref[0, :, 0] + deg_ref[1, :, 0]
    aggr = jnp.dot(agg, wm_ref[...], preferred_element_type=jnp.float32)
    aggr = aggr + deg[:, None] * bm_ref[...]
    h = (jnp.dot(x_ref[...], wa_ref[:D], preferred_element_type=jnp.float32)
         + jnp.dot(aggr, wa_ref[D:], preferred_element_type=jnp.float32)
         + ba_ref[...])
    h = jnp.maximum(h, 0.0)
    out_ref[...] = jnp.dot(h, wb_ref[...], preferred_element_type=jnp.float32) + bb_ref[...]


_tc_layer = pl.pallas_call(
    _layer_body,
    grid=(NB,),
    in_specs=[
        pl.BlockSpec((BN, D), lambda i: (i, 0)),
        pl.BlockSpec((NC, BN, D), lambda i: (0, i, 0)),
        pl.BlockSpec((NC, BN, 16), lambda i: (0, i, 0)),
        pl.BlockSpec((D, D), lambda i: (0, 0)),
        pl.BlockSpec((1, D), lambda i: (0, 0)),
        pl.BlockSpec((2 * D, D), lambda i: (0, 0)),
        pl.BlockSpec((1, D), lambda i: (0, 0)),
        pl.BlockSpec((D, D), lambda i: (0, 0)),
        pl.BlockSpec((1, D), lambda i: (0, 0)),
    ],
    out_specs=pl.BlockSpec((BN, D), lambda i: (i, 0)),
    out_shape=jax.ShapeDtypeStruct((N, D), jnp.float32),
)


def _layer_pool_body(x_ref, s_ref, deg_ref, wm_ref, bm_ref, wa_ref, ba_ref,
                     wb_ref, bb_ref, wo_ref, bo_ref, out_ref, acc_ref):
    agg = s_ref[0] + s_ref[1]
    deg = deg_ref[0, :, 0] + deg_ref[1, :, 0]
    aggr = jnp.dot(agg, wm_ref[...], preferred_element_type=jnp.float32)
    aggr = aggr + deg[:, None] * bm_ref[...]
    h = (jnp.dot(x_ref[...], wa_ref[:D], preferred_element_type=jnp.float32)
         + jnp.dot(aggr, wa_ref[D:], preferred_element_type=jnp.float32)
         + ba_ref[...])
    h = jnp.maximum(h, 0.0)
    y = jnp.dot(h, wb_ref[...], preferred_element_type=jnp.float32) + bb_ref[...]
    i = pl.program_id(0)

    @pl.when(i == 0)
    def _():
        acc_ref[...] = jnp.zeros_like(acc_ref)

    acc_ref[...] += jnp.sum(y, axis=0, keepdims=True)

    @pl.when(i == NB - 1)
    def _():
        pooled = acc_ref[...] * (1.0 / N)
        out_ref[...] = (jnp.dot(pooled, wo_ref[...],
                                preferred_element_type=jnp.float32)
                        + bo_ref[...])


_tc_layer_pool = pl.pallas_call(
    _layer_pool_body,
    grid=(NB,),
    in_specs=[
        pl.BlockSpec((BN, D), lambda i: (i, 0)),
        pl.BlockSpec((NC, BN, D), lambda i: (0, i, 0)),
        pl.BlockSpec((NC, BN, 16), lambda i: (0, i, 0)),
        pl.BlockSpec((D, D), lambda i: (0, 0)),
        pl.BlockSpec((1, D), lambda i: (0, 0)),
        pl.BlockSpec((2 * D, D), lambda i: (0, 0)),
        pl.BlockSpec((1, D), lambda i: (0, 0)),
        pl.BlockSpec((D, D), lambda i: (0, 0)),
        pl.BlockSpec((1, D), lambda i: (0, 0)),
        pl.BlockSpec((D, OUT), lambda i: (0, 0)),
        pl.BlockSpec((1, OUT), lambda i: (0, 0)),
    ],
    out_specs=pl.BlockSpec((1, OUT), lambda i: (0, 0)),
    out_shape=jax.ShapeDtypeStruct((1, OUT), jnp.float32),
    scratch_shapes=[pltpu.VMEM((1, OUT), jnp.float32)],
)


def kernel(x, edge_index, batch, W_m1, b_m1, W_u1a, b_u1a, W_u1b, b_u1b,
           W_m2, b_m2, W_u2a, b_u2a, W_u2b, b_u2b, W_out, b_out):
    src = edge_index[0]
    dst = edge_index[1]
    pad = EPAD - E
    pad_dst = N + (jnp.arange(pad, dtype=jnp.int32) % (NPAD - N))
    srcr = jnp.concatenate([src, jnp.zeros((pad,), jnp.int32)]).reshape(NT, KJ, CHUNK)
    dstr = jnp.concatenate([dst, pad_dst]).reshape(NT, KJ, CHUNK)
    zs = jnp.zeros((RPT, D), jnp.float32)
    zd = jnp.zeros((RPT, 16), jnp.float32)
    ones = jnp.ones((CHUNK, 16), jnp.float32)

    deg = _deg_count(dstr, zd, ones)
    s1 = _seg_sum(x, srcr, dstr, zs)
    h1 = _tc_layer(x, s1, deg, W_m1, b_m1.reshape(1, D), W_u1a,
                   b_u1a.reshape(1, D), W_u1b, b_u1b.reshape(1, D))
    s2 = _seg_sum(h1, srcr, dstr, zs)
    return _tc_layer_pool(h1, s2, deg, W_m2, b_m2.reshape(1, D), W_u2a,
                          b_u2a.reshape(1, D), W_u2b, b_u2b.reshape(1, D),
                          W_out, b_out.reshape(1, OUT))


# speedup vs baseline: 2.6270x; 2.6270x over previous
import functools

import jax
import jax.numpy as jnp
from jax import lax
from jax.experimental import pallas as pl
from jax.experimental.pallas import tpu as pltpu
from jax.experimental.pallas import tpu_sc as plsc

N = 10000
E = 320000
D = 128
OUT = 128

NC = 2
NS = 16
NT = NC * NS
CHUNK = 128
KJ = 80
NH = 2
HKJ = KJ // NH
EPAD = NT * KJ * CHUNK
NPAD = 10112
RPT = NPAD // NS

BN = 400
NB = N // BN

_mesh = plsc.VectorSubcoreMesh(core_axis_name="c", subcore_axis_name="s")


@functools.partial(
    pl.kernel,
    mesh=_mesh,
    out_type=jax.ShapeDtypeStruct((NC, NPAD, 16), jnp.float32),
    scratch_types=[
        pltpu.VMEM((KJ, CHUNK), jnp.int32),
        pltpu.VMEM((CHUNK, 16), jnp.float32),
        pltpu.VMEM_SHARED((NPAD, 16), jnp.float32),
    ],
)
def _deg_count(dst_hbm, zd_hbm, ones_hbm, d_out, idx_d, ones_v, d_sh):
    c = lax.axis_index("c")
    s = lax.axis_index("s")
    wid = c * NS + s
    pltpu.sync_copy(zd_hbm, d_sh.at[pl.ds(s * RPT, RPT)])
    pltpu.sync_copy(ones_hbm, ones_v)
    pltpu.sync_copy(dst_hbm.at[wid], idx_d)
    plsc.subcore_barrier()

    def body(j, carry):
        pltpu.sync_copy(ones_v, d_sh.at[idx_d.at[j]], add=True)
        return carry

    lax.fori_loop(0, KJ, body, 0)
    plsc.subcore_barrier()
    pltpu.sync_copy(d_sh.at[pl.ds(s * RPT, RPT)], d_out.at[c, pl.ds(s * RPT, RPT)])


@functools.partial(
    pl.kernel,
    mesh=_mesh,
    out_type=jax.ShapeDtypeStruct((NC, NPAD, D), jnp.float32),
    scratch_types=[
        pltpu.VMEM((KJ, CHUNK), jnp.int32),
        pltpu.VMEM((KJ, CHUNK), jnp.int32),
        pltpu.VMEM((CHUNK, D), jnp.float32),
        pltpu.VMEM_SHARED((NPAD, D), jnp.float32),
        pltpu.SemaphoreType.DMA,
    ],
)
def _seg_sum(x_hbm, src_hbm, dst_hbm, zs_hbm, s_out,
             idx_s, idx_d, rows, s_sh, sem):
    c = lax.axis_index("c")
    s = lax.axis_index("s")
    wid = c * NS + s

    pltpu.sync_copy(zs_hbm, s_sh.at[pl.ds(s * RPT, RPT)])
    pltpu.sync_copy(src_hbm.at[wid], idx_s)
    pltpu.sync_copy(dst_hbm.at[wid], idx_d)
    plsc.subcore_barrier()

    def body(j, carry):
        pltpu.async_copy(x_hbm.at[idx_s.at[j]], rows, sem).wait()
        pltpu.sync_copy(rows, s_sh.at[idx_d.at[j]], add=True)
        return carry

    lax.fori_loop(0, KJ, body, 0)
    plsc.subcore_barrier()
    pltpu.sync_copy(s_sh.at[pl.ds(s * RPT, RPT)], s_out.at[c, pl.ds(s * RPT, RPT)])


def _layer_body(x_ref, s_ref, deg_ref, wm_ref, bm_ref, wa_ref, ba_ref,
                wb_ref, bb_ref, out_ref):
    agg = s_ref[0] + s_ref[1]
    deg = deg_ref[0, :, 0] + deg_ref[1, :, 0]
    aggr = jnp.dot(agg, wm_ref[...], preferred_element_type=jnp.float32)
    aggr = aggr + deg[:, None] * bm_ref[...]
    h = (jnp.dot(x_ref[...], wa_ref[:D], preferred_element_type=jnp.float32)
         + jnp.dot(aggr, wa_ref[D:], preferred_element_type=jnp.float32)
         + ba_ref[...])
    h = jnp.maximum(h, 0.0)
    out_ref[...] = jnp.dot(h, wb_ref[...], preferred_element_type=jnp.float32) + bb_ref[...]


_tc_layer = pl.pallas_call(
    _layer_body,
    grid=(NB,),
    in_specs=[
        pl.BlockSpec((BN, D), lambda i: (i, 0)),
        pl.BlockSpec((NC, BN, D), lambda i: (0, i, 0)),
        pl.BlockSpec((NC, BN, 16), lambda i: (0, i, 0)),
        pl.BlockSpec((D, D), lambda i: (0, 0)),
        pl.BlockSpec((1, D), lambda i: (0, 0)),
        pl.BlockSpec((2 * D, D), lambda i: (0, 0)),
        pl.BlockSpec((1, D), lambda i: (0, 0)),
        pl.BlockSpec((D, D), lambda i: (0, 0)),
        pl.BlockSpec((1, D), lambda i: (0, 0)),
    ],
    out_specs=pl.BlockSpec((BN, D), lambda i: (i, 0)),
    out_shape=jax.ShapeDtypeStruct((N, D), jnp.float32),
)


def _layer_pool_body(x_ref, s_ref, deg_ref, wm_ref, bm_ref, wa_ref, ba_ref,
                     wb_ref, bb_ref, wo_ref, bo_ref, out_ref, acc_ref):
    agg = s_ref[0] + s_ref[1]
    deg = deg_ref[0, :, 0] + deg_ref[1, :, 0]
    aggr = jnp.dot(agg, wm_ref[...], preferred_element_type=jnp.float32)
    aggr = aggr + deg[:, None] * bm_ref[...]
    h = (jnp.dot(x_ref[...], wa_ref[:D], preferred_element_type=jnp.float32)
         + jnp.dot(aggr, wa_ref[D:], preferred_element_type=jnp.float32)
         + ba_ref[...])
    h = jnp.maximum(h, 0.0)
    y = jnp.dot(h, wb_ref[...], preferred_element_type=jnp.float32) + bb_ref[...]
    i = pl.program_id(0)

    @pl.when(i == 0)
    def _():
        acc_ref[...] = jnp.zeros_like(acc_ref)

    acc_ref[...] += jnp.sum(y, axis=0, keepdims=True)

    @pl.when(i == NB - 1)
    def _():
        pooled = acc_ref[...] * (1.0 / N)
        out_ref[...] = (jnp.dot(pooled, wo_ref[...],
                                preferred_element_type=jnp.float32)
                        + bo_ref[...])


_tc_layer_pool = pl.pallas_call(
    _layer_pool_body,
    grid=(NB,),
    in_specs=[
        pl.BlockSpec((BN, D), lambda i: (i, 0)),
        pl.BlockSpec((NC, BN, D), lambda i: (0, i, 0)),
        pl.BlockSpec((NC, BN, 16), lambda i: (0, i, 0)),
        pl.BlockSpec((D, D), lambda i: (0, 0)),
        pl.BlockSpec((1, D), lambda i: (0, 0)),
        pl.BlockSpec((2 * D, D), lambda i: (0, 0)),
        pl.BlockSpec((1, D), lambda i: (0, 0)),
        pl.BlockSpec((D, D), lambda i: (0, 0)),
        pl.BlockSpec((1, D), lambda i: (0, 0)),
        pl.BlockSpec((D, OUT), lambda i: (0, 0)),
        pl.BlockSpec((1, OUT), lambda i: (0, 0)),
    ],
    out_specs=pl.BlockSpec((1, OUT), lambda i: (0, 0)),
    out_shape=jax.ShapeDtypeStruct((1, OUT), jnp.float32),
    scratch_shapes=[pltpu.VMEM((1, OUT), jnp.float32)],
)


def kernel(x, edge_index, batch, W_m1, b_m1, W_u1a, b_u1a, W_u1b, b_u1b,
           W_m2, b_m2, W_u2a, b_u2a, W_u2b, b_u2b, W_out, b_out):
    src = edge_index[0]
    dst = edge_index[1]
    pad = EPAD - E
    pad_idx = jnp.arange(pad, dtype=jnp.int32)
    pad_dst = N + (pad_idx % (NPAD - N))
    pad_src = (pad_idx * 79) % N
    srcr = jnp.concatenate([src, pad_src]).reshape(NT, KJ, CHUNK)
    dstr = jnp.concatenate([dst, pad_dst]).reshape(NT, KJ, CHUNK)
    zs = jnp.zeros((RPT, D), jnp.float32)
    zd = jnp.zeros((RPT, 16), jnp.float32)
    ones = jnp.ones((CHUNK, 16), jnp.float32)

    deg = _deg_count(dstr, zd, ones)
    s1 = _seg_sum(x, srcr, dstr, zs)
    h1 = _tc_layer(x, s1, deg, W_m1, b_m1.reshape(1, D), W_u1a,
                   b_u1a.reshape(1, D), W_u1b, b_u1b.reshape(1, D))
    s2 = _seg_sum(h1, srcr, dstr, zs)
    return _tc_layer_pool(h1, s2, deg, W_m2, b_m2.reshape(1, D), W_u2a,
                          b_u2a.reshape(1, D), W_u2b, b_u2b.reshape(1, D),
                          W_out, b_out.reshape(1, OUT))


# speedup vs baseline: 3.0050x; 1.1439x over previous
import functools

import jax
import jax.numpy as jnp
from jax import lax
from jax.experimental import pallas as pl
from jax.experimental.pallas import tpu as pltpu
from jax.experimental.pallas import tpu_sc as plsc

N = 10000
E = 320000
D = 128
OUT = 128

NC = 2
NS = 16
NT = NC * NS
CHUNK = 128
KJ = 80
NH = 2
HKJ = KJ // NH
EPAD = NT * KJ * CHUNK
NPAD = 10112
RPT = NPAD // NS

BN = 400
NB = N // BN

_mesh = plsc.VectorSubcoreMesh(core_axis_name="c", subcore_axis_name="s")


@functools.partial(
    pl.kernel,
    mesh=_mesh,
    out_type=jax.ShapeDtypeStruct((NC, NPAD, 16), jnp.float32),
    scratch_types=[
        pltpu.VMEM((KJ, CHUNK), jnp.int32),
        pltpu.VMEM((CHUNK, 16), jnp.float32),
        pltpu.VMEM_SHARED((NPAD, 16), jnp.float32),
    ],
)
def _deg_count(dst_hbm, zd_hbm, ones_hbm, d_out, idx_d, ones_v, d_sh):
    c = lax.axis_index("c")
    s = lax.axis_index("s")
    wid = c * NS + s
    pltpu.sync_copy(zd_hbm, d_sh.at[pl.ds(s * RPT, RPT)])
    pltpu.sync_copy(ones_hbm, ones_v)
    pltpu.sync_copy(dst_hbm.at[wid], idx_d)
    plsc.subcore_barrier()

    def body(j, carry):
        pltpu.sync_copy(ones_v, d_sh.at[idx_d.at[j]], add=True)
        return carry

    lax.fori_loop(0, KJ, body, 0)
    plsc.subcore_barrier()
    pltpu.sync_copy(d_sh.at[pl.ds(s * RPT, RPT)], d_out.at[c, pl.ds(s * RPT, RPT)])


@functools.partial(
    pl.kernel,
    mesh=_mesh,
    out_type=jax.ShapeDtypeStruct((NC, NPAD, D), jnp.float32),
    scratch_types=[
        pltpu.VMEM((HKJ, CHUNK), jnp.int32),
        pltpu.VMEM((HKJ, CHUNK), jnp.int32),
        pltpu.VMEM((CHUNK, D), jnp.float32),
        pltpu.VMEM((CHUNK, D), jnp.float32),
        pltpu.VMEM_SHARED((NPAD, D), jnp.float32),
        pltpu.SemaphoreType.DMA,
        pltpu.SemaphoreType.DMA,
        pltpu.SemaphoreType.DMA,
        pltpu.SemaphoreType.DMA,
    ],
)
def _seg_sum(x_hbm, src_hbm, dst_hbm, zs_hbm, s_out,
             idx_s, idx_d, r0, r1, s_sh, g0, g1, w0, w1):
    c = lax.axis_index("c")
    s = lax.axis_index("s")
    wid = c * NS + s
    rows = (r0, r1)
    gsem = (g0, g1)
    ssem = (w0, w1)

    pltpu.sync_copy(zs_hbm, s_sh.at[pl.ds(s * RPT, RPT)])
    plsc.subcore_barrier()

    def issue_gather(j, b):
        pltpu.async_copy(x_hbm.at[idx_s.at[j]], rows[b], gsem[b])

    def wait_gather(j, b):
        pltpu.make_async_copy(x_hbm.at[idx_s.at[j]], rows[b], gsem[b]).wait()

    def issue_scatter(j, b):
        pltpu.async_copy(rows[b], s_sh.at[idx_d.at[j]], ssem[b], add=True)

    def wait_scatter(j, b):
        pltpu.make_async_copy(rows[b], s_sh.at[idx_d.at[j]], ssem[b]).wait()

    def half(h, carry):
        pltpu.sync_copy(src_hbm.at[wid, pl.ds(h * HKJ, HKJ)], idx_s)
        pltpu.sync_copy(dst_hbm.at[wid, pl.ds(h * HKJ, HKJ)], idx_d)
        issue_gather(0, 0)
        GMAX = HKJ // 2

        def body(g, carry2):
            j0 = 2 * g
            j1 = j0 + 1

            @pl.when(g >= 1)
            def _():
                wait_scatter(j1 - 2, 1)

            issue_gather(j1, 1)
            wait_gather(j0, 0)
            issue_scatter(j0, 0)
            wait_gather(j1, 1)
            issue_scatter(j1, 1)

            @pl.when(g < GMAX - 1)
            def _():
                wait_scatter(j0, 0)
                issue_gather(j0 + 2, 0)

            return carry2

        lax.fori_loop(0, GMAX, body, carry)
        wait_scatter(HKJ - 2, 0)
        wait_scatter(HKJ - 1, 1)
        return carry

    lax.fori_loop(0, NH, half, 0)
    plsc.subcore_barrier()
    pltpu.sync_copy(s_sh.at[pl.ds(s * RPT, RPT)], s_out.at[c, pl.ds(s * RPT, RPT)])


def _layer_body(x_ref, s_ref, deg_ref, wm_ref, bm_ref, wa_ref, ba_ref,
                wb_ref, bb_ref, out_ref):
    agg = s_ref[0] + s_ref[1]
    deg = deg_ref[0, :, 0] + deg_ref[1, :, 0]
    aggr = jnp.dot(agg, wm_ref[...], preferred_element_type=jnp.float32)
    aggr = aggr + deg[:, None] * bm_ref[...]
    h = (jnp.dot(x_ref[...], wa_ref[:D], preferred_element_type=jnp.float32)
         + jnp.dot(aggr, wa_ref[D:], preferred_element_type=jnp.float32)
         + ba_ref[...])
    h = jnp.maximum(h, 0.0)
    out_ref[...] = jnp.dot(h, wb_ref[...], preferred_element_type=jnp.float32) + bb_ref[...]


_tc_layer = pl.pallas_call(
    _layer_body,
    grid=(NB,),
    in_specs=[
        pl.BlockSpec((BN, D), lambda i: (i, 0)),
        pl.BlockSpec((NC, BN, D), lambda i: (0, i, 0)),
        pl.BlockSpec((NC, BN, 16), lambda i: (0, i, 0)),
        pl.BlockSpec((D, D), lambda i: (0, 0)),
        pl.BlockSpec((1, D), lambda i: (0, 0)),
        pl.BlockSpec((2 * D, D), lambda i: (0, 0)),
        pl.BlockSpec((1, D), lambda i: (0, 0)),
        pl.BlockSpec((D, D), lambda i: (0, 0)),
        pl.BlockSpec((1, D), lambda i: (0, 0)),
    ],
    out_specs=pl.BlockSpec((BN, D), lambda i: (i, 0)),
    out_shape=jax.ShapeDtypeStruct((N, D), jnp.float32),
)


def _layer_pool_body(x_ref, s_ref, deg_ref, wm_ref, bm_ref, wa_ref, ba_ref,
                     wb_ref, bb_ref, wo_ref, bo_ref, out_ref, acc_ref):
    agg = s_ref[0] + s_ref[1]
    deg = deg_ref[0, :, 0] + deg_ref[1, :, 0]
    aggr = jnp.dot(agg, wm_ref[...], preferred_element_type=jnp.float32)
    aggr = aggr + deg[:, None] * bm_ref[...]
    h = (jnp.dot(x_ref[...], wa_ref[:D], preferred_element_type=jnp.float32)
         + jnp.dot(aggr, wa_ref[D:], preferred_element_type=jnp.float32)
         + ba_ref[...])
    h = jnp.maximum(h, 0.0)
    y = jnp.dot(h, wb_ref[...], preferred_element_type=jnp.float32) + bb_ref[...]
    i = pl.program_id(0)

    @pl.when(i == 0)
    def _():
        acc_ref[...] = jnp.zeros_like(acc_ref)

    acc_ref[...] += jnp.sum(y, axis=0, keepdims=True)

    @pl.when(i == NB - 1)
    def _():
        pooled = acc_ref[...] * (1.0 / N)
        out_ref[...] = (jnp.dot(pooled, wo_ref[...],
                                preferred_element_type=jnp.float32)
                        + bo_ref[...])


_tc_layer_pool = pl.pallas_call(
    _layer_pool_body,
    grid=(NB,),
    in_specs=[
        pl.BlockSpec((BN, D), lambda i: (i, 0)),
        pl.BlockSpec((NC, BN, D), lambda i: (0, i, 0)),
        pl.BlockSpec((NC, BN, 16), lambda i: (0, i, 0)),
        pl.BlockSpec((D, D), lambda i: (0, 0)),
        pl.BlockSpec((1, D), lambda i: (0, 0)),
        pl.BlockSpec((2 * D, D), lambda i: (0, 0)),
        pl.BlockSpec((1, D), lambda i: (0, 0)),
        pl.BlockSpec((D, D), lambda i: (0, 0)),
        pl.BlockSpec((1, D), lambda i: (0, 0)),
        pl.BlockSpec((D, OUT), lambda i: (0, 0)),
        pl.BlockSpec((1, OUT), lambda i: (0, 0)),
    ],
    out_specs=pl.BlockSpec((1, OUT), lambda i: (0, 0)),
    out_shape=jax.ShapeDtypeStruct((1, OUT), jnp.float32),
    scratch_shapes=[pltpu.VMEM((1, OUT), jnp.float32)],
)


def kernel(x, edge_index, batch, W_m1, b_m1, W_u1a, b_u1a, W_u1b, b_u1b,
           W_m2, b_m2, W_u2a, b_u2a, W_u2b, b_u2b, W_out, b_out):
    src = edge_index[0]
    dst = edge_index[1]
    pad = EPAD - E
    pad_idx = jnp.arange(pad, dtype=jnp.int32)
    pad_dst = N + (pad_idx % (NPAD - N))
    pad_src = (pad_idx * 79) % N
    srcr = jnp.concatenate([src, pad_src]).reshape(NT, KJ, CHUNK)
    dstr = jnp.concatenate([dst, pad_dst]).reshape(NT, KJ, CHUNK)
    zs = jnp.zeros((RPT, D), jnp.float32)
    zd = jnp.zeros((RPT, 16), jnp.float32)
    ones = jnp.ones((CHUNK, 16), jnp.float32)

    deg = _deg_count(dstr, zd, ones)
    s1 = _seg_sum(x, srcr, dstr, zs)
    h1 = _tc_layer(x, s1, deg, W_m1, b_m1.reshape(1, D), W_u1a,
                   b_u1a.reshape(1, D), W_u1b, b_u1b.reshape(1, D))
    s2 = _seg_sum(h1, srcr, dstr, zs)
    return _tc_layer_pool(h1, s2, deg, W_m2, b_m2.reshape(1, D), W_u2a,
                          b_u2a.reshape(1, D), W_u2b, b_u2b.reshape(1, D),
                          W_out, b_out.reshape(1, OUT))


# speedup vs baseline: 3.1703x; 1.0550x over previous
import functools

import jax
import jax.numpy as jnp
from jax import lax
from jax.experimental import pallas as pl
from jax.experimental.pallas import tpu as pltpu
from jax.experimental.pallas import tpu_sc as plsc

N = 10000
E = 320000
D = 128
OUT = 128

NC = 2
NS = 16
NT = NC * NS
CHUNK = 128
KJ = 80
NH = 2
HKJ = KJ // NH
EPAD = NT * KJ * CHUNK
NPAD = 10112
RPT = NPAD // NS

BN = 1000
NB = N // BN

_mesh = plsc.VectorSubcoreMesh(core_axis_name="c", subcore_axis_name="s")


@functools.partial(
    pl.kernel,
    mesh=_mesh,
    out_type=jax.ShapeDtypeStruct((NC, NPAD, 16), jnp.float32),
    scratch_types=[
        pltpu.VMEM((KJ, CHUNK), jnp.int32),
        pltpu.VMEM((CHUNK, 16), jnp.float32),
        pltpu.VMEM_SHARED((NPAD, 16), jnp.float32),
        pltpu.SemaphoreType.DMA,
    ],
)
def _deg_count(dst_hbm, zd_hbm, ones_hbm, d_out, idx_d, ones_v, d_sh, sem):
    c = lax.axis_index("c")
    s = lax.axis_index("s")
    wid = c * NS + s
    pltpu.sync_copy(zd_hbm, d_sh.at[pl.ds(s * RPT, RPT)])
    pltpu.sync_copy(ones_hbm, ones_v)
    pltpu.sync_copy(dst_hbm.at[wid], idx_d)
    plsc.subcore_barrier()

    def body(j, carry):
        pltpu.async_copy(ones_v, d_sh.at[idx_d.at[j]], sem, add=True)
        return carry

    lax.fori_loop(0, KJ, body, 0)

    def drain(j, carry):
        pltpu.make_async_copy(ones_v, d_sh.at[idx_d.at[j]], sem).wait()
        return carry

    lax.fori_loop(0, KJ, drain, 0)
    plsc.subcore_barrier()
    pltpu.sync_copy(d_sh.at[pl.ds(s * RPT, RPT)], d_out.at[c, pl.ds(s * RPT, RPT)])


@functools.partial(
    pl.kernel,
    mesh=_mesh,
    out_type=jax.ShapeDtypeStruct((NC, NPAD, D), jnp.float32),
    scratch_types=[
        pltpu.VMEM((HKJ, CHUNK), jnp.int32),
        pltpu.VMEM((HKJ, CHUNK), jnp.int32),
        pltpu.VMEM((CHUNK, D), jnp.float32),
        pltpu.VMEM((CHUNK, D), jnp.float32),
        pltpu.VMEM_SHARED((NPAD, D), jnp.float32),
        pltpu.SemaphoreType.DMA,
        pltpu.SemaphoreType.DMA,
        pltpu.SemaphoreType.DMA,
        pltpu.SemaphoreType.DMA,
    ],
)
def _seg_sum(x_hbm, src_hbm, dst_hbm, zs_hbm, s_out,
             idx_s, idx_d, r0, r1, s_sh, g0, g1, w0, w1):
    c = lax.axis_index("c")
    s = lax.axis_index("s")
    wid = c * NS + s
    rows = (r0, r1)
    gsem = (g0, g1)
    ssem = (w0, w1)

    pltpu.sync_copy(zs_hbm, s_sh.at[pl.ds(s * RPT, RPT)])
    plsc.subcore_barrier()

    def issue_gather(j, b):
        pltpu.async_copy(x_hbm.at[idx_s.at[j]], rows[b], gsem[b])

    def wait_gather(j, b):
        pltpu.make_async_copy(x_hbm.at[idx_s.at[j]], rows[b], gsem[b]).wait()

    def issue_scatter(j, b):
        pltpu.async_copy(rows[b], s_sh.at[idx_d.at[j]], ssem[b], add=True)

    def wait_scatter(j, b):
        pltpu.make_async_copy(rows[b], s_sh.at[idx_d.at[j]], ssem[b]).wait()

    def half(h, carry):
        pltpu.sync_copy(src_hbm.at[wid, pl.ds(h * HKJ, HKJ)], idx_s)
        pltpu.sync_copy(dst_hbm.at[wid, pl.ds(h * HKJ, HKJ)], idx_d)
        issue_gather(0, 0)
        GMAX = HKJ // 2

        def body(g, carry2):
            j0 = 2 * g
            j1 = j0 + 1

            @pl.when(g >= 1)
            def _():
                wait_scatter(j1 - 2, 1)

            issue_gather(j1, 1)
            wait_gather(j0, 0)
            issue_scatter(j0, 0)
            wait_gather(j1, 1)
            issue_scatter(j1, 1)

            @pl.when(g < GMAX - 1)
            def _():
                wait_scatter(j0, 0)
                issue_gather(j0 + 2, 0)

            return carry2

        lax.fori_loop(0, GMAX, body, carry)
        wait_scatter(HKJ - 2, 0)
        wait_scatter(HKJ - 1, 1)
        return carry

    lax.fori_loop(0, NH, half, 0)
    plsc.subcore_barrier()
    pltpu.sync_copy(s_sh.at[pl.ds(s * RPT, RPT)], s_out.at[c, pl.ds(s * RPT, RPT)])


def _layer_body(x_ref, s_ref, deg_ref, wm_ref, bm_ref, wa_ref, ba_ref,
                wb_ref, bb_ref, out_ref):
    agg = s_ref[0] + s_ref[1]
    deg = deg_ref[0, :, 0] + deg_ref[1, :, 0]
    aggr = jnp.dot(agg, wm_ref[...], preferred_element_type=jnp.float32)
    aggr = aggr + deg[:, None] * bm_ref[...]
    h = (jnp.dot(x_ref[...], wa_ref[:D], preferred_element_type=jnp.float32)
         + jnp.dot(aggr, wa_ref[D:], preferred_element_type=jnp.float32)
         + ba_ref[...])
    h = jnp.maximum(h, 0.0)
    out_ref[...] = jnp.dot(h, wb_ref[...], preferred_element_type=jnp.float32) + bb_ref[...]


_tc_layer = pl.pallas_call(
    _layer_body,
    grid=(NB,),
    in_specs=[
        pl.BlockSpec((BN, D), lambda i: (i, 0)),
        pl.BlockSpec((NC, BN, D), lambda i: (0, i, 0)),
        pl.BlockSpec((NC, BN, 16), lambda i: (0, i, 0)),
        pl.BlockSpec((D, D), lambda i: (0, 0)),
        pl.BlockSpec((1, D), lambda i: (0, 0)),
        pl.BlockSpec((2 * D, D), lambda i: (0, 0)),
        pl.BlockSpec((1, D), lambda i: (0, 0)),
        pl.BlockSpec((D, D), lambda i: (0, 0)),
        pl.BlockSpec((1, D), lambda i: (0, 0)),
    ],
    out_specs=pl.BlockSpec((BN, D), lambda i: (i, 0)),
    out_shape=jax.ShapeDtypeStruct((N, D), jnp.float32),
)


def _layer_pool_body(x_ref, s_ref, deg_ref, wm_ref, bm_ref, wa_ref, ba_ref,
                     wb_ref, bb_ref, wo_ref, bo_ref, out_ref, acc_ref):
    agg = s_ref[0] + s_ref[1]
    deg = deg_ref[0, :, 0] + deg_ref[1, :, 0]
    aggr = jnp.dot(agg, wm_ref[...], preferred_element_type=jnp.float32)
    aggr = aggr + deg[:, None] * bm_ref[...]
    h = (jnp.dot(x_ref[...], wa_ref[:D], preferred_element_type=jnp.float32)
         + jnp.dot(aggr, wa_ref[D:], preferred_element_type=jnp.float32)
         + ba_ref[...])
    h = jnp.maximum(h, 0.0)
    y = jnp.dot(h, wb_ref[...], preferred_element_type=jnp.float32) + bb_ref[...]
    i = pl.program_id(0)

    @pl.when(i == 0)
    def _():
        acc_ref[...] = jnp.zeros_like(acc_ref)

    acc_ref[...] += jnp.sum(y, axis=0, keepdims=True)

    @pl.when(i == NB - 1)
    def _():
        pooled = acc_ref[...] * (1.0 / N)
        out_ref[...] = (jnp.dot(pooled, wo_ref[...],
                                preferred_element_type=jnp.float32)
                        + bo_ref[...])


_tc_layer_pool = pl.pallas_call(
    _layer_pool_body,
    grid=(NB,),
    in_specs=[
        pl.BlockSpec((BN, D), lambda i: (i, 0)),
        pl.BlockSpec((NC, BN, D), lambda i: (0, i, 0)),
        pl.BlockSpec((NC, BN, 16), lambda i: (0, i, 0)),
        pl.BlockSpec((D, D), lambda i: (0, 0)),
        pl.BlockSpec((1, D), lambda i: (0, 0)),
        pl.BlockSpec((2 * D, D), lambda i: (0, 0)),
        pl.BlockSpec((1, D), lambda i: (0, 0)),
        pl.BlockSpec((D, D), lambda i: (0, 0)),
        pl.BlockSpec((1, D), lambda i: (0, 0)),
        pl.BlockSpec((D, OUT), lambda i: (0, 0)),
        pl.BlockSpec((1, OUT), lambda i: (0, 0)),
    ],
    out_specs=pl.BlockSpec((1, OUT), lambda i: (0, 0)),
    out_shape=jax.ShapeDtypeStruct((1, OUT), jnp.float32),
    scratch_shapes=[pltpu.VMEM((1, OUT), jnp.float32)],
)


def kernel(x, edge_index, batch, W_m1, b_m1, W_u1a, b_u1a, W_u1b, b_u1b,
           W_m2, b_m2, W_u2a, b_u2a, W_u2b, b_u2b, W_out, b_out):
    src = edge_index[0]
    dst = edge_index[1]
    pad = EPAD - E
    pad_idx = jnp.arange(pad, dtype=jnp.int32)
    pad_dst = N + (pad_idx % (NPAD - N))
    pad_src = (pad_idx * 79) % N
    srcr = jnp.concatenate([src, pad_src]).reshape(NT, KJ, CHUNK)
    dstr = jnp.concatenate([dst, pad_dst]).reshape(NT, KJ, CHUNK)
    zs = jnp.zeros((RPT, D), jnp.float32)
    zd = jnp.zeros((RPT, 16), jnp.float32)
    ones = jnp.ones((CHUNK, 16), jnp.float32)

    deg = _deg_count(dstr, zd, ones)
    s1 = _seg_sum(x, srcr, dstr, zs)
    h1 = _tc_layer(x, s1, deg, W_m1, b_m1.reshape(1, D), W_u1a,
                   b_u1a.reshape(1, D), W_u1b, b_u1b.reshape(1, D))
    s2 = _seg_sum(h1, srcr, dstr, zs)
    return _tc_layer_pool(h1, s2, deg, W_m2, b_m2.reshape(1, D), W_u2a,
                          b_u2a.reshape(1, D), W_u2b, b_u2b.reshape(1, D),
                          W_out, b_out.reshape(1, OUT))
